# baseline (speedup 1.0000x reference)
import functools

import jax
import jax.numpy as jnp
from jax import lax
from jax.experimental import pallas as pl
from jax.experimental.pallas import tpu as pltpu
from jax.experimental.pallas import tpu_sc as plsc

NC = 2
NS = 16
D = 128
W = 128
NR = 2
NI = 8

_SIZES = [(50000, 20000), (20000, 8000), (8000, 2048)]


def _sc_segment_sum(x, src, dst, n_dst, dst_split):
    n_edges = src.shape[0]
    half = n_dst // 2 if dst_split else n_dst
    pad = ((half + 129 + 1023) // 1024) * 1024
    per_sub_zero = pad // NS
    chunk = n_edges // NS if dst_split else n_edges // (NC * NS)
    nwin = chunk // W
    assert nwin * W == chunk and nwin % NI == 0 and nwin >= NI
    out_rows = (half // NS) // 8 * 8
    out_rem = half - NS * out_rows
    mesh = plsc.VectorSubcoreMesh(core_axis_name="c", subcore_axis_name="s")

    z128 = jnp.zeros((16, D), jnp.float32)
    z1d = jnp.zeros((per_sub_zero,), jnp.float32)
    ones = jnp.ones((W,), jnp.float32)

    if dst_split:
        out_type = (jax.ShapeDtypeStruct((n_dst, D), jnp.float32),
                    jax.ShapeDtypeStruct((n_dst,), jnp.float32))
    else:
        out_type = (jax.ShapeDtypeStruct((NC, n_dst, D), jnp.float32),
                    jax.ShapeDtypeStruct((NC * n_dst,), jnp.float32))

    scratch = (
        [pltpu.VMEM((W,), jnp.int32) for _ in range(NI)]
        + [pltpu.VMEM((W,), jnp.int32) for _ in range(NI)]
        + [pltpu.VMEM((W, D), jnp.float32) for _ in range(NR)]
        + [pltpu.VMEM((16, D), jnp.float32),
           pltpu.VMEM((W,), jnp.float32),
           pltpu.VMEM((per_sub_zero,), jnp.float32),
           pltpu.VMEM_SHARED((pad, D), jnp.float32),
           pltpu.VMEM_SHARED((pad,), jnp.float32)]
        + [pltpu.SemaphoreType.DMA for _ in range(NI)]
        + [pltpu.SemaphoreType.DMA for _ in range(3 * NR)]
    )

    @functools.partial(pl.kernel, out_type=out_type, mesh=mesh,
                       scratch_types=scratch)
    def k(x_hbm, src_hbm, dst_hbm, z128_hbm, z1d_hbm, ones_hbm,
          sums_hbm, cnt_hbm, *scr):
        srcw = scr[0:NI]
        dstw = scr[NI:2 * NI]
        rows = scr[2 * NI:2 * NI + NR]
        zblk, ovec, cvec, accum, cacc = scr[2 * NI + NR:2 * NI + NR + 5]
        rest = scr[2 * NI + NR + 5:]
        s_i = rest[0:NI]
        s_g = rest[NI:NI + NR]
        s_sr = rest[NI + NR:NI + 2 * NR]
        s_sc = rest[NI + 2 * NR:NI + 3 * NR]

        c = lax.axis_index("c")
        s = lax.axis_index("s")
        lo = c * half if dst_split else 0
        pltpu.sync_copy(z128_hbm, zblk)
        pltpu.sync_copy(ones_hbm, ovec)

        zb = s * per_sub_zero
        pltpu.sync_copy(z1d_hbm, cvec)
        pltpu.sync_copy(cvec, cacc.at[pl.ds(zb, per_sub_zero)])

        @pl.loop(0, per_sub_zero, step=16)
        def _(r):
            pltpu.sync_copy(zblk, accum.at[pl.ds(zb + r, 16)])

        plsc.subcore_barrier()

        base = s * chunk if dst_split else (c * NS + s) * chunk

        def fire_idx(i, b):
            eb = base + i * W
            pltpu.async_copy(src_hbm.at[pl.ds(eb, W)], srcw[b], s_i[b])
            pltpu.async_copy(dst_hbm.at[pl.ds(eb, W)], dstw[b], s_i[b])

        def wait_idx(b):
            pltpu.make_async_copy(src_hbm.at[pl.ds(0, W)], srcw[b],
                                  s_i[b]).wait()
            pltpu.make_async_copy(dst_hbm.at[pl.ds(0, W)], dstw[b],
                                  s_i[b]).wait()

        def fire_gather(b8, b4):
            pltpu.async_copy(x_hbm.at[srcw[b8]], rows[b4], s_g[b4])

        def wait_gather(b8, b4):
            pltpu.make_async_copy(x_hbm.at[srcw[b8]], rows[b4],
                                  s_g[b4]).wait()

        def fire_scatter(b8, b4):
            pltpu.async_copy(rows[b4], accum.at[dstw[b8]], s_sr[b4],
                             add=True)
            pltpu.async_copy(ovec, cacc.at[dstw[b8]], s_sc[b4], add=True)

        def wait_scatter(b8, b4):
            pltpu.make_async_copy(rows[b4], accum.at[dstw[b8]],
                                  s_sr[b4]).wait()
            pltpu.make_async_copy(ovec, cacc.at[dstw[b8]], s_sc[b4]).wait()

        def remap(b, i):
            db = dstw[b]
            trash_base = half + ((s + i) % 8) * 16

            @pl.loop(0, W, step=16)
            def _(j):
                d = db[pl.ds(j, 16)]
                local = d - lo
                oob = (local < 0) | (local >= half)
                trash = trash_base + lax.iota(jnp.int32, 16)
                db[pl.ds(j, 16)] = jnp.where(oob, trash, local)

        for i in range(5):
            fire_idx(i, i % NI)

        @pl.loop(0, nwin, step=NI)
        def _(wi):
            for off in range(NI):
                i = wi + off
                wait_idx(off)

                @pl.when(i >= 2)
                def _():
                    wait_scatter((off - 2) % NI, (off - 2) % NR)

                fire_gather(off, off % NR)
                if dst_split:
                    remap(off, i)
                wait_gather(off, off % NR)
                fire_scatter(off, off % NR)

                @pl.when(i + 5 < nwin)
                def _():
                    fire_idx(i + 5, (off + 5) % NI)

        for t in (2, 1):
            wait_scatter((nwin - t) % NI, (nwin - t) % NR)

        plsc.subcore_barrier()
        rw = s * out_rows
        if dst_split:
            sums_slice = lambda a, n: sums_hbm.at[pl.ds(lo + a, n)]
            cnt_off = lo
        else:
            sums_slice = lambda a, n: sums_hbm.at[c, pl.ds(a, n)]
            cnt_off = c * n_dst
        pltpu.sync_copy(accum.at[pl.ds(rw, out_rows)],
                        sums_slice(rw, out_rows))
        pltpu.sync_copy(cacc.at[pl.ds(rw, out_rows)],
                        cvec.at[pl.ds(0, out_rows)])
        pltpu.sync_copy(cvec.at[pl.ds(0, out_rows)],
                        cnt_hbm.at[pl.ds(cnt_off + rw, out_rows)])
        if out_rem:
            @pl.when(s == 0)
            def _():
                rr = NS * out_rows
                pltpu.sync_copy(accum.at[pl.ds(rr, out_rem)],
                                sums_slice(rr, out_rem))
                pltpu.sync_copy(cacc.at[pl.ds(rr, out_rem)],
                                cvec.at[pl.ds(0, out_rem)])
                pltpu.sync_copy(cvec.at[pl.ds(0, out_rem)],
                                cnt_hbm.at[pl.ds(cnt_off + rr, out_rem)])

    return k(x, src, dst, z128, z1d, ones)


def _tc_combine(sums, cnt, x_dst, Wl, Wr, b, relu):
    n_dst = x_dst.shape[0]
    partial = sums.ndim == 3
    br = 2000 if n_dst % 2000 == 0 else (1024 if n_dst % 1024 == 0 else n_dst)
    b2 = b.reshape(1, D)

    def body(s_ref, c_ref, xd_ref, wl_ref, wr_ref, b_ref, o_ref):
        if partial:
            s = s_ref[0] + s_ref[1]
            ctot = c_ref[0] + c_ref[1]
        else:
            s = s_ref[...]
            ctot = c_ref[...]
        mean = s / jnp.maximum(ctot, 1.0)
        o = (jnp.dot(mean, wl_ref[...], preferred_element_type=jnp.float32)
             + jnp.dot(xd_ref[...], wr_ref[...],
                       preferred_element_type=jnp.float32)
             + b_ref[...])
        if relu:
            o = jnp.maximum(o, 0.0)
        o_ref[...] = o

    if partial:
        s_spec = pl.BlockSpec((NC, br, D), lambda i: (0, i, 0))
        c_spec = pl.BlockSpec((NC, br, 1), lambda i: (0, i, 0))
    else:
        s_spec = pl.BlockSpec((br, D), lambda i: (i, 0))
        c_spec = pl.BlockSpec((br, 1), lambda i: (i, 0))

    return pl.pallas_call(
        body,
        grid=(n_dst // br,),
        in_specs=[
            s_spec,
            c_spec,
            pl.BlockSpec((br, D), lambda i: (i, 0)),
            pl.BlockSpec((D, D), lambda i: (0, 0)),
            pl.BlockSpec((D, D), lambda i: (0, 0)),
            pl.BlockSpec((1, D), lambda i: (0, 0)),
        ],
        out_specs=pl.BlockSpec((br, D), lambda i: (i, 0)),
        out_shape=jax.ShapeDtypeStruct((n_dst, D), jnp.float32),
    )(sums, cnt, x_dst, Wl, Wr, b2)


def _pad_edges(ei, n_dst, n_total):
    src, dst = ei[0], ei[1]
    n = src.shape[0]
    if n == n_total:
        return src, dst
    extra = n_total - n
    src = jnp.concatenate([src, jnp.zeros((extra,), jnp.int32)])
    pad_rows = n_dst + (jnp.arange(extra, dtype=jnp.int32) % 64)
    dst = jnp.concatenate([dst, pad_rows])
    return src, dst


def kernel(x, edge_index0, edge_index1, edge_index2,
           Wl0, Wr0, b0, Wl1, Wr1, b1, Wl2, Wr2, b2):
    edges = [edge_index0, edge_index1, edge_index2]
    params = [(Wl0, Wr0, b0), (Wl1, Wr1, b1), (Wl2, Wr2, b2)]
    h = x
    for i, (ei, (ns, nd)) in enumerate(zip(edges, _SIZES)):
        dst_split = i == 0
        n_edges = ei.shape[1]
        workers = NS if dst_split else NC * NS
        quantum = workers * NI * W
        n_total = ((n_edges + quantum - 1) // quantum) * quantum
        src, dst = _pad_edges(ei, nd, n_total)
        Wl, Wr, b = params[i]
        sums, cnt = _sc_segment_sum(h, src, dst, nd, dst_split)
        if dst_split:
            cnt = cnt.reshape(nd, 1)
        else:
            cnt = cnt.reshape(NC, nd, 1)
        h = _tc_combine(sums, cnt, h[:nd], Wl, Wr, b, relu=(i != 2))
    return h


# speedup vs baseline: 1.1595x; 1.1595x over previous
import functools

import jax
import jax.numpy as jnp
from jax import lax
from jax.experimental import pallas as pl
from jax.experimental.pallas import tpu as pltpu
from jax.experimental.pallas import tpu_sc as plsc

NC = 2
NS = 16
D = 128
NR = 2
NI = 4

_SIZES = [(50000, 20000), (20000, 8000), (8000, 2048)]


def _sc_segment_sum(x, src, dst, n_dst, w, dst_split):
    n_edges = src.shape[0]
    half = n_dst // 2 if dst_split else n_dst
    pad = ((half + 129 + 1023) // 1024) * 1024
    per_sub_zero = pad // NS
    chunk = n_edges // NS if dst_split else n_edges // (NC * NS)
    nwin = chunk // w
    assert nwin * w == chunk and nwin % NI == 0 and nwin >= NI
    out_rows = (half // NS) // 8 * 8
    out_rem = half - NS * out_rows
    mesh = plsc.VectorSubcoreMesh(core_axis_name="c", subcore_axis_name="s")

    z128 = jnp.zeros((16, D), jnp.float32)
    z1d = jnp.zeros((per_sub_zero,), jnp.float32)
    ones = jnp.ones((w,), jnp.float32)

    if dst_split:
        out_type = (jax.ShapeDtypeStruct((n_dst, D), jnp.float32),
                    jax.ShapeDtypeStruct((n_dst,), jnp.float32))
    else:
        out_type = (jax.ShapeDtypeStruct((NC, n_dst, D), jnp.float32),
                    jax.ShapeDtypeStruct((NC * n_dst,), jnp.float32))

    scratch = (
        [pltpu.VMEM((w,), jnp.int32) for _ in range(NI)]
        + [pltpu.VMEM((w,), jnp.int32) for _ in range(NI)]
        + [pltpu.VMEM((w, D), jnp.float32) for _ in range(NR)]
        + [pltpu.VMEM((16, D), jnp.float32),
           pltpu.VMEM((w,), jnp.float32),
           pltpu.VMEM((per_sub_zero,), jnp.float32),
           pltpu.VMEM_SHARED((pad, D), jnp.float32),
           pltpu.VMEM_SHARED((pad,), jnp.float32)]
        + [pltpu.SemaphoreType.DMA for _ in range(NI)]
        + [pltpu.SemaphoreType.DMA for _ in range(3 * NR)]
    )

    @functools.partial(pl.kernel, out_type=out_type, mesh=mesh,
                       scratch_types=scratch)
    def k(x_hbm, src_hbm, dst_hbm, z128_hbm, z1d_hbm, ones_hbm,
          sums_hbm, cnt_hbm, *scr):
        srcw = scr[0:NI]
        dstw = scr[NI:2 * NI]
        rows = scr[2 * NI:2 * NI + NR]
        zblk, ovec, cvec, accum, cacc = scr[2 * NI + NR:2 * NI + NR + 5]
        rest = scr[2 * NI + NR + 5:]
        s_i = rest[0:NI]
        s_g = rest[NI:NI + NR]
        s_sr = rest[NI + NR:NI + 2 * NR]
        s_sc = rest[NI + 2 * NR:NI + 3 * NR]

        c = lax.axis_index("c")
        s = lax.axis_index("s")
        lo = c * half if dst_split else 0
        pltpu.sync_copy(z128_hbm, zblk)
        pltpu.sync_copy(ones_hbm, ovec)

        zb = s * per_sub_zero
        pltpu.sync_copy(z1d_hbm, cvec)
        pltpu.sync_copy(cvec, cacc.at[pl.ds(zb, per_sub_zero)])

        @pl.loop(0, per_sub_zero, step=16)
        def _(r):
            pltpu.sync_copy(zblk, accum.at[pl.ds(zb + r, 16)])

        plsc.subcore_barrier()

        base = s * chunk if dst_split else (c * NS + s) * chunk

        def fire_idx(i, b):
            eb = base + i * w
            pltpu.async_copy(src_hbm.at[pl.ds(eb, w)], srcw[b], s_i[b])
            pltpu.async_copy(dst_hbm.at[pl.ds(eb, w)], dstw[b], s_i[b])

        def wait_idx(b):
            pltpu.make_async_copy(src_hbm.at[pl.ds(0, w)], srcw[b],
                                  s_i[b]).wait()
            pltpu.make_async_copy(dst_hbm.at[pl.ds(0, w)], dstw[b],
                                  s_i[b]).wait()

        def fire_gather(b8, b4):
            pltpu.async_copy(x_hbm.at[srcw[b8]], rows[b4], s_g[b4])

        def wait_gather(b8, b4):
            pltpu.make_async_copy(x_hbm.at[srcw[b8]], rows[b4],
                                  s_g[b4]).wait()

        def fire_scatter(b8, b4):
            pltpu.async_copy(rows[b4], accum.at[dstw[b8]], s_sr[b4],
                             add=True)
            pltpu.async_copy(ovec, cacc.at[dstw[b8]], s_sc[b4], add=True)

        def wait_scatter(b8, b4):
            pltpu.make_async_copy(rows[b4], accum.at[dstw[b8]],
                                  s_sr[b4]).wait()
            pltpu.make_async_copy(ovec, cacc.at[dstw[b8]], s_sc[b4]).wait()

        def remap(b, i):
            db = dstw[b]
            trash_base = half + ((s + i) % 8) * 16

            @pl.loop(0, w, step=16)
            def _(j):
                d = db[pl.ds(j, 16)]
                local = d - lo
                oob = (local < 0) | (local >= half)
                trash = trash_base + lax.iota(jnp.int32, 16)
                db[pl.ds(j, 16)] = jnp.where(oob, trash, local)

        for i in range(2):
            fire_idx(i, i % NI)

        @pl.loop(0, nwin, step=NI)
        def _(wi):
            for off in range(NI):
                i = wi + off
                wait_idx(off)

                @pl.when(i >= 2)
                def _():
                    wait_scatter((off - 2) % NI, (off - 2) % NR)

                fire_gather(off, off % NR)
                if dst_split:
                    remap(off, i)
                wait_gather(off, off % NR)
                fire_scatter(off, off % NR)

                @pl.when(i + 2 < nwin)
                def _():
                    fire_idx(i + 2, (off + 2) % NI)

        for t in (2, 1):
            wait_scatter((nwin - t) % NI, (nwin - t) % NR)

        plsc.subcore_barrier()
        rw = s * out_rows
        if dst_split:
            sums_slice = lambda a, n: sums_hbm.at[pl.ds(lo + a, n)]
            cnt_off = lo
        else:
            sums_slice = lambda a, n: sums_hbm.at[c, pl.ds(a, n)]
            cnt_off = c * n_dst
        pltpu.sync_copy(accum.at[pl.ds(rw, out_rows)],
                        sums_slice(rw, out_rows))
        pltpu.sync_copy(cacc.at[pl.ds(rw, out_rows)],
                        cvec.at[pl.ds(0, out_rows)])
        pltpu.sync_copy(cvec.at[pl.ds(0, out_rows)],
                        cnt_hbm.at[pl.ds(cnt_off + rw, out_rows)])
        if out_rem:
            @pl.when(s == 0)
            def _():
                rr = NS * out_rows
                pltpu.sync_copy(accum.at[pl.ds(rr, out_rem)],
                                sums_slice(rr, out_rem))
                pltpu.sync_copy(cacc.at[pl.ds(rr, out_rem)],
                                cvec.at[pl.ds(0, out_rem)])
                pltpu.sync_copy(cvec.at[pl.ds(0, out_rem)],
                                cnt_hbm.at[pl.ds(cnt_off + rr, out_rem)])

    return k(x, src, dst, z128, z1d, ones)


def _tc_combine(sums, cnt, x_dst, Wl, Wr, b, relu):
    n_dst = x_dst.shape[0]
    partial = sums.ndim == 3
    br = 2000 if n_dst % 2000 == 0 else (1024 if n_dst % 1024 == 0 else n_dst)
    b2 = b.reshape(1, D)

    def body(s_ref, c_ref, xd_ref, wl_ref, wr_ref, b_ref, o_ref):
        if partial:
            s = s_ref[0] + s_ref[1]
            ctot = c_ref[0] + c_ref[1]
        else:
            s = s_ref[...]
            ctot = c_ref[...]
        mean = s / jnp.maximum(ctot, 1.0)
        o = (jnp.dot(mean, wl_ref[...], preferred_element_type=jnp.float32)
             + jnp.dot(xd_ref[...], wr_ref[...],
                       preferred_element_type=jnp.float32)
             + b_ref[...])
        if relu:
            o = jnp.maximum(o, 0.0)
        o_ref[...] = o

    if partial:
        s_spec = pl.BlockSpec((NC, br, D), lambda i: (0, i, 0))
        c_spec = pl.BlockSpec((NC, br, 1), lambda i: (0, i, 0))
    else:
        s_spec = pl.BlockSpec((br, D), lambda i: (i, 0))
        c_spec = pl.BlockSpec((br, 1), lambda i: (i, 0))

    return pl.pallas_call(
        body,
        grid=(n_dst // br,),
        in_specs=[
            s_spec,
            c_spec,
            pl.BlockSpec((br, D), lambda i: (i, 0)),
            pl.BlockSpec((D, D), lambda i: (0, 0)),
            pl.BlockSpec((D, D), lambda i: (0, 0)),
            pl.BlockSpec((1, D), lambda i: (0, 0)),
        ],
        out_specs=pl.BlockSpec((br, D), lambda i: (i, 0)),
        out_shape=jax.ShapeDtypeStruct((n_dst, D), jnp.float32),
    )(sums, cnt, x_dst, Wl, Wr, b2)


def _pad_edges(ei, n_dst, n_total):
    src, dst = ei[0], ei[1]
    n = src.shape[0]
    if n == n_total:
        return src, dst
    extra = n_total - n
    src = jnp.concatenate([src, jnp.zeros((extra,), jnp.int32)])
    pad_rows = n_dst + (jnp.arange(extra, dtype=jnp.int32) % 64)
    dst = jnp.concatenate([dst, pad_rows])
    return src, dst


def kernel(x, edge_index0, edge_index1, edge_index2,
           Wl0, Wr0, b0, Wl1, Wr1, b1, Wl2, Wr2, b2):
    edges = [edge_index0, edge_index1, edge_index2]
    params = [(Wl0, Wr0, b0), (Wl1, Wr1, b1), (Wl2, Wr2, b2)]
    ws = [80, 80, 64]
    h = x
    for i, (ei, (ns, nd)) in enumerate(zip(edges, _SIZES)):
        dst_split = i == 0
        n_edges = ei.shape[1]
        workers = NS if dst_split else NC * NS
        quantum = workers * NI * ws[i]
        n_total = ((n_edges + quantum - 1) // quantum) * quantum
        src, dst = _pad_edges(ei, nd, n_total)
        Wl, Wr, b = params[i]
        sums, cnt = _sc_segment_sum(h, src, dst, nd, ws[i], dst_split)
        if dst_split:
            cnt = cnt.reshape(nd, 1)
        else:
            cnt = cnt.reshape(NC, nd, 1)
        h = _tc_combine(sums, cnt, h[:nd], Wl, Wr, b, relu=(i != 2))
    return h


# speedup vs baseline: 2.2519x; 1.9421x over previous
import functools

import jax
import jax.numpy as jnp
from jax import lax
from jax.experimental import pallas as pl
from jax.experimental.pallas import tpu as pltpu
from jax.experimental.pallas import tpu_sc as plsc

NC = 2
NS = 16
D = 128

_SIZES = [(50000, 20000), (20000, 8000), (8000, 2048)]


def _sc_segment_sum(x, src, dst, n_dst, w, dst_split):
    n_edges = src.shape[0]
    half = n_dst // 2 if dst_split else n_dst
    pad = ((half + (128 if dst_split else 0) + 1023) // 1024) * 1024
    per_sub_zero = pad // NS
    chunk = n_edges // NS if dst_split else n_edges // (NC * NS)
    nwin = chunk // w
    assert nwin % 2 == 0 and nwin >= 2
    out_rows = (half // NS) // 8 * 8
    out_rem = half - NS * out_rows
    mesh = plsc.VectorSubcoreMesh(core_axis_name="c", subcore_axis_name="s")

    z128 = jnp.zeros((64, D), jnp.float32)
    z1d = jnp.zeros((per_sub_zero,), jnp.float32)
    ones = jnp.ones((w,), jnp.float32)

    if dst_split:
        out_type = (jax.ShapeDtypeStruct((n_dst, D), jnp.float32),
                    jax.ShapeDtypeStruct((n_dst,), jnp.float32))
    else:
        out_type = (jax.ShapeDtypeStruct((NC, n_dst, D), jnp.float32),
                    jax.ShapeDtypeStruct((NC * n_dst,), jnp.float32))

    @functools.partial(
        pl.kernel,
        out_type=out_type,
        mesh=mesh,
        scratch_types=[
            pltpu.VMEM((w,), jnp.int32),
            pltpu.VMEM((w,), jnp.int32),
            pltpu.VMEM((w, D), jnp.float32),
            pltpu.VMEM((w,), jnp.int32),
            pltpu.VMEM((w,), jnp.int32),
            pltpu.VMEM((w, D), jnp.float32),
            pltpu.VMEM((64, D), jnp.float32),
            pltpu.VMEM((w,), jnp.float32),
            pltpu.VMEM((per_sub_zero,), jnp.float32),
            pltpu.VMEM_SHARED((pad, D), jnp.float32),
            pltpu.VMEM_SHARED((pad,), jnp.float32),
            pltpu.SemaphoreType.DMA,
            pltpu.SemaphoreType.DMA,
            pltpu.SemaphoreType.DMA,
            pltpu.SemaphoreType.DMA,
            pltpu.SemaphoreType.DMA,
            pltpu.SemaphoreType.DMA,
            pltpu.SemaphoreType.DMA,
            pltpu.SemaphoreType.DMA,
        ],
    )
    def k(x_hbm, src_hbm, dst_hbm, z128_hbm, z1d_hbm, ones_hbm,
          sums_hbm, cnt_hbm,
          srcw0, dstw0, rows0, srcw1, dstw1, rows1, zblk, ovec, cvec,
          accum, cacc,
          s_i0, s_i1, s_g0, s_g1, s_sr0, s_sr1, s_sc0, s_sc1):
        c = lax.axis_index("c")
        s = lax.axis_index("s")
        lo = c * half if dst_split else 0
        pltpu.sync_copy(z128_hbm, zblk)
        pltpu.sync_copy(ones_hbm, ovec)

        zb = s * per_sub_zero
        pltpu.sync_copy(z1d_hbm, cvec)
        pltpu.sync_copy(cvec, cacc.at[pl.ds(zb, per_sub_zero)])

        @pl.loop(0, per_sub_zero, step=64)
        def _(r):
            pltpu.sync_copy(zblk, accum.at[pl.ds(zb + r, 64)])

        plsc.subcore_barrier()

        base = s * chunk if dst_split else (c * NS + s) * chunk
        bufs = [(srcw0, dstw0, rows0, s_i0, s_g0, s_sr0, s_sc0),
                (srcw1, dstw1, rows1, s_i1, s_g1, s_sr1, s_sc1)]

        def fire_idx(i, sb, db, si):
            eb = base + i * w
            pltpu.async_copy(src_hbm.at[pl.ds(eb, w)], sb, si)
            pltpu.async_copy(dst_hbm.at[pl.ds(eb, w)], db, si)

        def wait_idx(sb, db, si):
            pltpu.make_async_copy(src_hbm.at[pl.ds(0, w)], sb, si).wait()
            pltpu.make_async_copy(dst_hbm.at[pl.ds(0, w)], db, si).wait()

        def fire_scatter(rb, db, ssr, ssc):
            pltpu.async_copy(rb, accum.at[db], ssr, add=True)
            pltpu.async_copy(ovec, cacc.at[db], ssc, add=True)

        def wait_scatter(rb, db, ssr, ssc):
            pltpu.make_async_copy(rb, accum.at[db], ssr).wait()
            pltpu.make_async_copy(ovec, cacc.at[db], ssc).wait()

        def remap(db, i):
            trash_base = half + ((s + i) % 8) * 16

            @pl.loop(0, w, step=16)
            def _(j):
                d = db[pl.ds(j, 16)]
                local = d - lo
                oob = (local < 0) | (local >= half)
                trash = trash_base + lax.iota(jnp.int32, 16)
                db[pl.ds(j, 16)] = jnp.where(oob, trash, local)

        fire_idx(0, srcw0, dstw0, s_i0)

        @pl.loop(0, nwin, step=2)
        def _(wi):
            for off in (0, 1):
                sb, db, rb, si, sg, ssr, ssc = bufs[off]
                osb, odb, orb, osi, osg, ossr, ossc = bufs[1 - off]
                i = wi + off
                wait_idx(sb, db, si)
                gd = pltpu.async_copy(x_hbm.at[sb], rb, sg)
                if dst_split:
                    remap(db, i)

                @pl.when(i > 0)
                def _():
                    wait_scatter(orb, odb, ossr, ossc)

                @pl.when(i + 1 < nwin)
                def _():
                    fire_idx(i + 1, osb, odb, osi)

                gd.wait()
                fire_scatter(rb, db, ssr, ssc)

        wait_scatter(rows1, dstw1, s_sr1, s_sc1)

        plsc.subcore_barrier()
        rw = s * out_rows
        if dst_split:
            sums_slice = lambda a, n: sums_hbm.at[pl.ds(lo + a, n)]
            cnt_off = lo
        else:
            sums_slice = lambda a, n: sums_hbm.at[c, pl.ds(a, n)]
            cnt_off = c * n_dst
        pltpu.sync_copy(accum.at[pl.ds(rw, out_rows)],
                        sums_slice(rw, out_rows))
        pltpu.sync_copy(cacc.at[pl.ds(rw, out_rows)],
                        cvec.at[pl.ds(0, out_rows)])
        pltpu.sync_copy(cvec.at[pl.ds(0, out_rows)],
                        cnt_hbm.at[pl.ds(cnt_off + rw, out_rows)])
        if out_rem:
            @pl.when(s == 0)
            def _():
                rr = NS * out_rows
                pltpu.sync_copy(accum.at[pl.ds(rr, out_rem)],
                                sums_slice(rr, out_rem))
                pltpu.sync_copy(cacc.at[pl.ds(rr, out_rem)],
                                cvec.at[pl.ds(0, out_rem)])
                pltpu.sync_copy(cvec.at[pl.ds(0, out_rem)],
                                cnt_hbm.at[pl.ds(cnt_off + rr, out_rem)])

    return k(x, src, dst, z128, z1d, ones)


def _tc_combine(sums, cnt, x_dst, Wl, Wr, b, relu):
    n_dst = x_dst.shape[0]
    partial = sums.ndim == 3
    br = 2000 if n_dst % 2000 == 0 else (1024 if n_dst % 1024 == 0 else n_dst)
    b2 = b.reshape(1, D)

    def body(s_ref, c_ref, xd_ref, wl_ref, wr_ref, b_ref, o_ref):
        if partial:
            s = s_ref[0] + s_ref[1]
            ctot = c_ref[0] + c_ref[1]
        else:
            s = s_ref[...]
            ctot = c_ref[...]
        mean = s / jnp.maximum(ctot, 1.0)
        o = (jnp.dot(mean, wl_ref[...], preferred_element_type=jnp.float32)
             + jnp.dot(xd_ref[...], wr_ref[...],
                       preferred_element_type=jnp.float32)
             + b_ref[...])
        if relu:
            o = jnp.maximum(o, 0.0)
        o_ref[...] = o

    if partial:
        s_spec = pl.BlockSpec((NC, br, D), lambda i: (0, i, 0))
        c_spec = pl.BlockSpec((NC, br, 1), lambda i: (0, i, 0))
    else:
        s_spec = pl.BlockSpec((br, D), lambda i: (i, 0))
        c_spec = pl.BlockSpec((br, 1), lambda i: (i, 0))

    return pl.pallas_call(
        body,
        grid=(n_dst // br,),
        in_specs=[
            s_spec,
            c_spec,
            pl.BlockSpec((br, D), lambda i: (i, 0)),
            pl.BlockSpec((D, D), lambda i: (0, 0)),
            pl.BlockSpec((D, D), lambda i: (0, 0)),
            pl.BlockSpec((1, D), lambda i: (0, 0)),
        ],
        out_specs=pl.BlockSpec((br, D), lambda i: (i, 0)),
        out_shape=jax.ShapeDtypeStruct((n_dst, D), jnp.float32),
    )(sums, cnt, x_dst, Wl, Wr, b2)


def kernel(x, edge_index0, edge_index1, edge_index2,
           Wl0, Wr0, b0, Wl1, Wr1, b1, Wl2, Wr2, b2):
    edges = [edge_index0, edge_index1, edge_index2]
    params = [(Wl0, Wr0, b0), (Wl1, Wr1, b1), (Wl2, Wr2, b2)]
    ws = [80, 80, 64]
    h = x
    for i, (ei, (ns, nd)) in enumerate(zip(edges, _SIZES)):
        src, dst = ei[0], ei[1]
        Wl, Wr, b = params[i]
        sums, cnt = _sc_segment_sum(h, src, dst, nd, ws[i], dst_split=(i == 0))
        if i == 0:
            cnt = cnt.reshape(nd, 1)
        else:
            cnt = cnt.reshape(NC, nd, 1)
        h = _tc_combine(sums, cnt, h[:nd], Wl, Wr, b, relu=(i != 2))
    return h
